# baseline (speedup 1.0000x reference)
import functools
import jax
import jax.numpy as jnp
from jax import lax
from jax.experimental import pallas as pl
from jax.experimental.pallas import tpu as pltpu
from jax.experimental.pallas import tpu_sc as plsc

_N = 10000
_NP = 10240
_E = 320000
_NT = 16
_RPT = _NP // _NT
_K = 128
_CPT = 160
_CPG = _NT * _CPT
_EP = _CPG * _K
_NPAIR = _CPT // 2

_mesh = plsc.VectorSubcoreMesh(core_axis_name="c", subcore_axis_name="s")


@functools.partial(
    pl.kernel,
    out_type=jax.ShapeDtypeStruct((2 * _NP,), jnp.float32),
    mesh=_mesh,
    scratch_types=[
        pltpu.VMEM((_K,), jnp.int32),
        pltpu.VMEM((_RPT,), jnp.float32),
        pltpu.VMEM_SHARED((_NP,), jnp.float32),
    ],
)
def _deg(dst_hbm, deg_hbm, dst_v, ones_v, acc):
    c = lax.axis_index("c")
    s = lax.axis_index("s")

    def fill(i, carry):
        ones_v[pl.ds(i * 16, 16)] = jnp.full((16,), 1.0, jnp.float32)
        return carry

    lax.fori_loop(0, _RPT // 16, fill, 0)
    pltpu.sync_copy(ones_v, acc.at[pl.ds(s * _RPT, _RPT)])
    plsc.subcore_barrier()

    ebase = (c * _CPG + s * _CPT) * _K

    def body(i, carry):
        pltpu.sync_copy(dst_hbm.at[pl.ds(ebase + i * _K, _K)], dst_v)
        pltpu.sync_copy(ones_v.at[pl.ds(0, _K)], acc.at[dst_v], add=True)
        return carry

    lax.fori_loop(0, _CPT, body, 0)
    plsc.subcore_barrier()
    pltpu.sync_copy(acc.at[pl.ds(s * _RPT, _RPT)], ones_v)
    pltpu.sync_copy(ones_v, deg_hbm.at[pl.ds(c * _NP + s * _RPT, _RPT)])


_NPAIRS = _CPT // 2


_NQUAD = _CPT // 4


@functools.partial(
    pl.kernel,
    out_type=jax.ShapeDtypeStruct((2 * _NP, 128), jnp.float32),
    mesh=_mesh,
    scratch_types=[
        pltpu.VMEM((_K,), jnp.int32),
        pltpu.VMEM((_K,), jnp.int32),
        pltpu.VMEM((_K,), jnp.int32),
        pltpu.VMEM((_K,), jnp.int32),
        pltpu.VMEM((_K,), jnp.int32),
        pltpu.VMEM((_K,), jnp.int32),
        pltpu.VMEM((_K,), jnp.int32),
        pltpu.VMEM((_K,), jnp.int32),
        pltpu.VMEM((_K, 128), jnp.float32),
        pltpu.VMEM((_K, 128), jnp.float32),
        pltpu.VMEM_SHARED((_NP, 128), jnp.float32),
        pltpu.SemaphoreType.DMA,
        pltpu.SemaphoreType.DMA,
        pltpu.SemaphoreType.DMA,
        pltpu.SemaphoreType.DMA,
        pltpu.SemaphoreType.DMA,
        pltpu.SemaphoreType.DMA,
        pltpu.SemaphoreType.DMA,
        pltpu.SemaphoreType.DMA,
    ],
)
def _agg(hp_hbm, src_hbm, dst_hbm, out_hbm, src0, dst0, src1, dst1,
         src2, dst2, src3, dst3, rows0, rows1, acc,
         isem0, isem1, isem2, isem3, gsem0, gsem1, ssem0, ssem1):
    c = lax.axis_index("c")
    s = lax.axis_index("s")
    row0 = s * _RPT
    ebase = (c * _CPG + s * _CPT) * _K
    for k in range(_RPT // _K):
        pltpu.sync_copy(hp_hbm.at[pl.ds(c * _NP + row0 + k * _K, _K)], rows0)
        pltpu.sync_copy(rows0, acc.at[pl.ds(row0 + k * _K, _K)])
    plsc.subcore_barrier()

    srcs = (src0, src1, src2, src3)
    dsts = (dst0, dst1, dst2, dst3)
    rows = (rows0, rows1)
    isems = (isem0, isem1, isem2, isem3)
    gsems = (gsem0, gsem1)
    ssems = (ssem0, ssem1)

    def _idx_load(i, p, sync=False):
        if sync:
            pltpu.sync_copy(src_hbm.at[pl.ds(ebase + i * _K, _K)], srcs[p])
            pltpu.sync_copy(dst_hbm.at[pl.ds(ebase + i * _K, _K)], dsts[p])
        else:
            pltpu.async_copy(src_hbm.at[pl.ds(ebase + i * _K, _K)],
                             srcs[p], isems[p])
            pltpu.async_copy(dst_hbm.at[pl.ds(ebase + i * _K, _K)],
                             dsts[p], isems[p])

    def _idx_wait(i, p):
        pltpu.make_async_copy(src_hbm.at[pl.ds(ebase + i * _K, _K)],
                              srcs[p], isems[p]).wait()
        pltpu.make_async_copy(dst_hbm.at[pl.ds(ebase + i * _K, _K)],
                              dsts[p], isems[p]).wait()

    def _scat_wait(r, p):
        pltpu.make_async_copy(rows[r], acc.at[dsts[p]], ssems[r]).wait()

    def _chunk(j, p, first_quad, last_quad):
        i = 4 * j + p
        r = p % 2
        pltpu.make_async_copy(hp_hbm.at[srcs[p]], rows[r], gsems[r]).wait()
        pltpu.async_copy(rows[r], acc.at[dsts[p]], ssems[r], add=True)
        if not (last_quad and p >= 2):
            _idx_load(i + 2, (p + 2) % 4)
        if not (last_quad and p == 3):
            _idx_wait(i + 1, (p + 1) % 4)
            if not (first_quad and p == 0):
                _scat_wait(1 - r, (p + 3) % 4)
            pltpu.async_copy(hp_hbm.at[srcs[(p + 1) % 4]], rows[1 - r],
                             gsems[1 - r])
        else:
            _scat_wait(1 - r, (p + 3) % 4)

    _idx_load(0, 0, sync=True)
    _idx_load(1, 1)
    pltpu.async_copy(hp_hbm.at[srcs[0]], rows[0], gsems[0])

    def quad(j, carry):
        for p in range(4):
            _chunk(j, p, False, False)
        return carry

    for p in range(4):
        _chunk(0, p, True, False)
    lax.fori_loop(1, _NQUAD - 1, quad, 0)
    for p in range(4):
        _chunk(_NQUAD - 1, p, False, True)
    _scat_wait(1, 3)

    plsc.subcore_barrier()
    for k in range(_RPT // _K):
        pltpu.sync_copy(acc.at[pl.ds(row0 + k * _K, _K)], rows0)
        pltpu.sync_copy(rows0, out_hbm.at[pl.ds(c * _NP + row0 + k * _K,
                                                _K)])


_BM = 2048


def _tc1(x, W, deg):
    def body(x_ref, w_ref, deg_ref, hp_ref, dinv_ref):
        d = lax.rsqrt(deg_ref[...])
        h = jnp.dot(x_ref[...], w_ref[...], preferred_element_type=jnp.float32)
        hp_ref[...] = d * h
        dinv_ref[...] = d

    return pl.pallas_call(
        body,
        grid=(2 * _NP // _BM,),
        in_specs=[
            pl.BlockSpec((_BM, 128), lambda i: (i, 0)),
            pl.BlockSpec((128, 128), lambda i: (0, 0)),
            pl.BlockSpec((_BM, 1), lambda i: (i, 0)),
        ],
        out_specs=[
            pl.BlockSpec((_BM, 128), lambda i: (i, 0)),
            pl.BlockSpec((_BM, 1), lambda i: (i, 0)),
        ],
        out_shape=[
            jax.ShapeDtypeStruct((2 * _NP, 128), jnp.float32),
            jax.ShapeDtypeStruct((2 * _NP, 1), jnp.float32),
        ],
    )(x, W, deg)


def _tc2(agg, dinv, b, W):
    F = b.shape[1]

    def body(a_ref, d_ref, b_ref, w_ref, o_ref):
        d = d_ref[...]
        y = jnp.maximum(d * a_ref[:, :F] + b_ref[...], 0.0)
        o_ref[...] = d * jnp.dot(y, w_ref[...],
                                 preferred_element_type=jnp.float32)

    return pl.pallas_call(
        body,
        grid=(2 * _NP // _BM,),
        in_specs=[
            pl.BlockSpec((_BM, 128), lambda i: (i, 0)),
            pl.BlockSpec((_BM, 1), lambda i: (i, 0)),
            pl.BlockSpec((1, F), lambda i: (0, 0)),
            pl.BlockSpec((F, 128), lambda i: (0, 0)),
        ],
        out_specs=pl.BlockSpec((_BM, 128), lambda i: (i, 0)),
        out_shape=jax.ShapeDtypeStruct((2 * _NP, 128), jnp.float32),
    )(agg, dinv, b, W)


def _head(agg3, dinv, b3, att_W, ntn_Wt, ntn_Vt, ntn_bR, f1w, f1b, f2w, f2b,
          f3w, f3b, scw, scb, av):
    def body(a_ref, d_ref, b3_ref, aw_ref, nw_ref, nv_ref, nb_ref, f1w_ref,
             f1b_ref, f2w_ref, f2b_ref, f3w_ref, f3b_ref, scw_ref, scb_ref,
             av_ref, score_ref, ged_ref):
        ps = []
        for g in range(2):
            y = (d_ref[pl.ds(g * _NP, _N), :]
                 * a_ref[pl.ds(g * _NP, _N), :32] + b3_ref[...])
            t1 = jnp.dot(y, aw_ref[...], preferred_element_type=jnp.float32)
            gc = jnp.sum(t1, axis=0, keepdims=True) * (1.0 / _N)
            tg = jnp.tanh(gc)
            sall = jax.nn.sigmoid(jnp.sum(y * tg, axis=1, keepdims=True))
            ps.append(jnp.sum(y * sall, axis=0, keepdims=True))
        p1, p2 = ps
        sc_list = []
        for t in range(16):
            m = jnp.sum(nw_ref[t] * p2, axis=1, keepdims=True)
            sc_list.append(jnp.dot(p1, m, preferred_element_type=jnp.float32))
        scoring = jnp.concatenate(sc_list, axis=1)
        combined = jnp.concatenate([p1, p2], axis=1)
        block = jnp.dot(combined, nv_ref[...],
                        preferred_element_type=jnp.float32)
        scores = jnp.maximum(scoring + block + nb_ref[...], 0.0)
        h = jnp.maximum(jnp.dot(scores, f1w_ref[...],
                                preferred_element_type=jnp.float32)
                        + f1b_ref[...], 0.0)
        h = jnp.maximum(jnp.dot(h, f2w_ref[...],
                                preferred_element_type=jnp.float32)
                        + f2b_ref[...], 0.0)
        h = jnp.maximum(jnp.dot(h, f3w_ref[...],
                                preferred_element_type=jnp.float32)
                        + f3b_ref[...], 0.0)
        sc = jax.nn.sigmoid(jnp.dot(h, scw_ref[...],
                                    preferred_element_type=jnp.float32)
                            + scb_ref[...])
        score_ref[...] = sc
        ged_ref[...] = -jnp.log(sc) * av_ref[...]

    return pl.pallas_call(
        body,
        out_shape=[
            jax.ShapeDtypeStruct((1, 1), jnp.float32),
            jax.ShapeDtypeStruct((1, 1), jnp.float32),
        ],
    )(agg3, dinv, b3, att_W, ntn_Wt, ntn_Vt, ntn_bR, f1w, f1b, f2w, f2b,
      f3w, f3b, scw, scb, av)


def kernel(features_1, features_2, edge_index_1, edge_index_2, avg_v,
           W1, b1, W2, b2, W3, b3, att_W, ntn_W, ntn_V, ntn_b,
           fc1_W, fc1_b, fc2_W, fc2_b, fc3_W, fc3_b, sc_W, sc_b):
    pad = jnp.zeros((_NP - _N, 128), jnp.float32)
    x = jnp.concatenate([features_1, pad, features_2, pad], axis=0)
    epad = jnp.full((_EP - _E,), _N, jnp.int32) + (
        jnp.arange(_EP - _E, dtype=jnp.int32) % (_NP - _N))
    src = jnp.concatenate([edge_index_1[0], epad,
                           edge_index_2[0] + _NP, epad + _NP])
    dst = jnp.concatenate([edge_index_1[1], epad, edge_index_2[1], epad])

    deg = _deg(dst).reshape(2 * _NP, 1)

    W2p = jnp.pad(W2, ((0, 0), (0, 128 - W2.shape[1])))
    W3p = jnp.pad(W3, ((0, 0), (0, 128 - W3.shape[1])))

    hp1, dinv = _tc1(x, W1, deg)
    agg1 = _agg(hp1, src, dst)
    hp2 = _tc2(agg1, dinv, b1.reshape(1, -1), W2p)
    agg2 = _agg(hp2, src, dst)
    hp3 = _tc2(agg2, dinv, b2.reshape(1, -1), W3p)
    agg3 = _agg(hp3, src, dst)

    score, ged = _head(
        agg3, dinv, b3.reshape(1, -1), att_W,
        jnp.transpose(ntn_W, (2, 0, 1)), ntn_V.T, ntn_b.reshape(1, -1),
        fc1_W, fc1_b.reshape(1, -1), fc2_W, fc2_b.reshape(1, -1),
        fc3_W, fc3_b.reshape(1, -1), sc_W, sc_b.reshape(1, -1),
        avg_v.reshape(1, 1))
    return score.reshape(-1), ged.reshape(-1)


# speedup vs baseline: 1.1446x; 1.1446x over previous
import functools
import jax
import jax.numpy as jnp
from jax import lax
from jax.experimental import pallas as pl
from jax.experimental.pallas import tpu as pltpu
from jax.experimental.pallas import tpu_sc as plsc

_N = 10000
_NP = 10240
_E = 320000
_NT = 16
_RPT = _NP // _NT
_K = 128
_CPT = 160
_CPG = _NT * _CPT
_EP = _CPG * _K
_NPAIR = _CPT // 2

_mesh = plsc.VectorSubcoreMesh(core_axis_name="c", subcore_axis_name="s")


@functools.partial(
    pl.kernel,
    out_type=jax.ShapeDtypeStruct((2 * _NP,), jnp.float32),
    mesh=_mesh,
    scratch_types=[
        pltpu.VMEM((_CPT, _K), jnp.int32),
        pltpu.VMEM((_RPT,), jnp.float32),
        pltpu.VMEM_SHARED((_NP,), jnp.float32),
    ],
)
def _deg(dst_hbm, deg_hbm, dst_blk, ones_v, acc):
    c = lax.axis_index("c")
    s = lax.axis_index("s")

    def fill(i, carry):
        ones_v[pl.ds(i * 16, 16)] = jnp.full((16,), 1.0, jnp.float32)
        return carry

    lax.fori_loop(0, _RPT // 16, fill, 0)
    pltpu.sync_copy(dst_hbm.at[pl.ds(c * _CPG + s * _CPT, _CPT)], dst_blk)
    pltpu.sync_copy(ones_v, acc.at[pl.ds(s * _RPT, _RPT)])
    plsc.subcore_barrier()

    def body(i, carry):
        pltpu.sync_copy(ones_v.at[pl.ds(0, _K)], acc.at[dst_blk.at[i]],
                        add=True)
        return carry

    lax.fori_loop(0, _CPT, body, 0)
    plsc.subcore_barrier()
    pltpu.sync_copy(acc.at[pl.ds(s * _RPT, _RPT)], ones_v)
    pltpu.sync_copy(ones_v, deg_hbm.at[pl.ds(c * _NP + s * _RPT, _RPT)])


_NPAIRS = _CPT // 2


@functools.partial(
    pl.kernel,
    out_type=jax.ShapeDtypeStruct((2 * _NP, 128), jnp.float32),
    mesh=_mesh,
    scratch_types=[
        pltpu.VMEM((_K,), jnp.int32),
        pltpu.VMEM((_K,), jnp.int32),
        pltpu.VMEM((_K,), jnp.int32),
        pltpu.VMEM((_K,), jnp.int32),
        pltpu.VMEM((_K, 128), jnp.float32),
        pltpu.VMEM((_K, 128), jnp.float32),
        pltpu.VMEM_SHARED((_NP, 128), jnp.float32),
        pltpu.SemaphoreType.DMA,
        pltpu.SemaphoreType.DMA,
        pltpu.SemaphoreType.DMA,
        pltpu.SemaphoreType.DMA,
    ],
)
def _agg(hp_hbm, src_hbm, dst_hbm, out_hbm, src0, dst0, src1, dst1,
         rows0, rows1, acc, isem0, isem1, gsem0, gsem1):
    c = lax.axis_index("c")
    s = lax.axis_index("s")
    row0 = s * _RPT
    ebase = (c * _CPG + s * _CPT) * _K
    for k in range(_RPT // _K):
        pltpu.sync_copy(hp_hbm.at[pl.ds(c * _NP + row0 + k * _K, _K)], rows0)
        pltpu.sync_copy(rows0, acc.at[pl.ds(row0 + k * _K, _K)])
    plsc.subcore_barrier()

    srcs = (src0, src1)
    dsts = (dst0, dst1)
    rows = (rows0, rows1)
    isems = (isem0, isem1)
    gsems = (gsem0, gsem1)

    def _idx_load(i, p, sync=False):
        if sync:
            pltpu.sync_copy(src_hbm.at[pl.ds(ebase + i * _K, _K)], srcs[p])
            pltpu.sync_copy(dst_hbm.at[pl.ds(ebase + i * _K, _K)], dsts[p])
        else:
            pltpu.async_copy(src_hbm.at[pl.ds(ebase + i * _K, _K)],
                             srcs[p], isems[p])
            pltpu.async_copy(dst_hbm.at[pl.ds(ebase + i * _K, _K)],
                             dsts[p], isems[p])

    def _idx_wait(i, p):
        pltpu.make_async_copy(src_hbm.at[pl.ds(ebase + i * _K, _K)],
                              srcs[p], isems[p]).wait()
        pltpu.make_async_copy(dst_hbm.at[pl.ds(ebase + i * _K, _K)],
                              dsts[p], isems[p]).wait()

    def _chunk(i, p, last_pair):
        if not last_pair or p == 0:
            _idx_wait(i + 1, 1 - p)
            pltpu.async_copy(hp_hbm.at[srcs[1 - p]], rows[1 - p],
                             gsems[1 - p])
        pltpu.make_async_copy(hp_hbm.at[srcs[p]], rows[p], gsems[p]).wait()
        pltpu.sync_copy(rows[p], acc.at[dsts[p]], add=True)
        if not last_pair:
            _idx_load(i + 2, p)

    _idx_load(0, 0, sync=True)
    pltpu.async_copy(hp_hbm.at[srcs[0]], rows[0], gsems[0])
    _idx_load(1, 1)

    def pair(j, carry):
        _chunk(2 * j, 0, False)
        _chunk(2 * j + 1, 1, False)
        return carry

    lax.fori_loop(0, _NPAIRS - 1, pair, 0)
    _chunk(_CPT - 2, 0, True)
    _chunk(_CPT - 1, 1, True)

    plsc.subcore_barrier()
    for k in range(_RPT // _K):
        pltpu.sync_copy(acc.at[pl.ds(row0 + k * _K, _K)], rows0)
        pltpu.sync_copy(rows0, out_hbm.at[pl.ds(c * _NP + row0 + k * _K,
                                                _K)])


_BM = 2048


def _tc1(x, W, deg):
    def body(x_ref, w_ref, deg_ref, hp_ref, dinv_ref):
        d = lax.rsqrt(deg_ref[...])
        h = jnp.dot(x_ref[...], w_ref[...], preferred_element_type=jnp.float32)
        hp_ref[...] = d * h
        dinv_ref[...] = d

    return pl.pallas_call(
        body,
        grid=(2 * _NP // _BM,),
        in_specs=[
            pl.BlockSpec((_BM, 128), lambda i: (i, 0)),
            pl.BlockSpec((128, 128), lambda i: (0, 0)),
            pl.BlockSpec((_BM, 1), lambda i: (i, 0)),
        ],
        out_specs=[
            pl.BlockSpec((_BM, 128), lambda i: (i, 0)),
            pl.BlockSpec((_BM, 1), lambda i: (i, 0)),
        ],
        out_shape=[
            jax.ShapeDtypeStruct((2 * _NP, 128), jnp.float32),
            jax.ShapeDtypeStruct((2 * _NP, 1), jnp.float32),
        ],
    )(x, W, deg)


def _tc2(agg, dinv, b, W):
    F = b.shape[1]

    def body(a_ref, d_ref, b_ref, w_ref, o_ref):
        d = d_ref[...]
        y = jnp.maximum(d * a_ref[:, :F] + b_ref[...], 0.0)
        o_ref[...] = d * jnp.dot(y, w_ref[...],
                                 preferred_element_type=jnp.float32)

    return pl.pallas_call(
        body,
        grid=(2 * _NP // _BM,),
        in_specs=[
            pl.BlockSpec((_BM, 128), lambda i: (i, 0)),
            pl.BlockSpec((_BM, 1), lambda i: (i, 0)),
            pl.BlockSpec((1, F), lambda i: (0, 0)),
            pl.BlockSpec((F, 128), lambda i: (0, 0)),
        ],
        out_specs=pl.BlockSpec((_BM, 128), lambda i: (i, 0)),
        out_shape=jax.ShapeDtypeStruct((2 * _NP, 128), jnp.float32),
    )(agg, dinv, b, W)


def _head(agg3, dinv, b3, att_W, ntn_Wt, ntn_Vt, ntn_bR, f1w, f1b, f2w, f2b,
          f3w, f3b, scw, scb, av):
    def body(a_ref, d_ref, b3_ref, aw_ref, nw_ref, nv_ref, nb_ref, f1w_ref,
             f1b_ref, f2w_ref, f2b_ref, f3w_ref, f3b_ref, scw_ref, scb_ref,
             av_ref, score_ref, ged_ref):
        ps = []
        for g in range(2):
            y = (d_ref[pl.ds(g * _NP, _N), :]
                 * a_ref[pl.ds(g * _NP, _N), :32] + b3_ref[...])
            t1 = jnp.dot(y, aw_ref[...], preferred_element_type=jnp.float32)
            gc = jnp.sum(t1, axis=0, keepdims=True) * (1.0 / _N)
            tg = jnp.tanh(gc)
            sall = jax.nn.sigmoid(jnp.sum(y * tg, axis=1, keepdims=True))
            ps.append(jnp.sum(y * sall, axis=0, keepdims=True))
        p1, p2 = ps
        sc_list = []
        for t in range(16):
            m = jnp.sum(nw_ref[t] * p2, axis=1, keepdims=True)
            sc_list.append(jnp.dot(p1, m, preferred_element_type=jnp.float32))
        scoring = jnp.concatenate(sc_list, axis=1)
        combined = jnp.concatenate([p1, p2], axis=1)
        block = jnp.dot(combined, nv_ref[...],
                        preferred_element_type=jnp.float32)
        scores = jnp.maximum(scoring + block + nb_ref[...], 0.0)
        h = jnp.maximum(jnp.dot(scores, f1w_ref[...],
                                preferred_element_type=jnp.float32)
                        + f1b_ref[...], 0.0)
        h = jnp.maximum(jnp.dot(h, f2w_ref[...],
                                preferred_element_type=jnp.float32)
                        + f2b_ref[...], 0.0)
        h = jnp.maximum(jnp.dot(h, f3w_ref[...],
                                preferred_element_type=jnp.float32)
                        + f3b_ref[...], 0.0)
        sc = jax.nn.sigmoid(jnp.dot(h, scw_ref[...],
                                    preferred_element_type=jnp.float32)
                            + scb_ref[...])
        score_ref[...] = sc
        ged_ref[...] = -jnp.log(sc) * av_ref[...]

    return pl.pallas_call(
        body,
        out_shape=[
            jax.ShapeDtypeStruct((1, 1), jnp.float32),
            jax.ShapeDtypeStruct((1, 1), jnp.float32),
        ],
    )(agg3, dinv, b3, att_W, ntn_Wt, ntn_Vt, ntn_bR, f1w, f1b, f2w, f2b,
      f3w, f3b, scw, scb, av)


def kernel(features_1, features_2, edge_index_1, edge_index_2, avg_v,
           W1, b1, W2, b2, W3, b3, att_W, ntn_W, ntn_V, ntn_b,
           fc1_W, fc1_b, fc2_W, fc2_b, fc3_W, fc3_b, sc_W, sc_b):
    pad = jnp.zeros((_NP - _N, 128), jnp.float32)
    x = jnp.concatenate([features_1, pad, features_2, pad], axis=0)
    epad = jnp.full((_EP - _E,), _N, jnp.int32) + (
        jnp.arange(_EP - _E, dtype=jnp.int32) % (_NP - _N))
    src = jnp.concatenate([edge_index_1[0], epad,
                           edge_index_2[0] + _NP, epad + _NP])
    dst = jnp.concatenate([edge_index_1[1], epad, edge_index_2[1], epad])

    deg = _deg(dst.reshape(2 * _CPG, _K)).reshape(2 * _NP, 1)

    W2p = jnp.pad(W2, ((0, 0), (0, 128 - W2.shape[1])))
    W3p = jnp.pad(W3, ((0, 0), (0, 128 - W3.shape[1])))

    hp1, dinv = _tc1(x, W1, deg)
    agg1 = _agg(hp1, src, dst)
    hp2 = _tc2(agg1, dinv, b1.reshape(1, -1), W2p)
    agg2 = _agg(hp2, src, dst)
    hp3 = _tc2(agg2, dinv, b2.reshape(1, -1), W3p)
    agg3 = _agg(hp3, src, dst)

    score, ged = _head(
        agg3, dinv, b3.reshape(1, -1), att_W,
        jnp.transpose(ntn_W, (2, 0, 1)), ntn_V.T, ntn_b.reshape(1, -1),
        fc1_W, fc1_b.reshape(1, -1), fc2_W, fc2_b.reshape(1, -1),
        fc3_W, fc3_b.reshape(1, -1), sc_W, sc_b.reshape(1, -1),
        avg_v.reshape(1, 1))
    return score.reshape(-1), ged.reshape(-1)


# speedup vs baseline: 1.2542x; 1.0957x over previous
import functools
import jax
import jax.numpy as jnp
from jax import lax
from jax.experimental import pallas as pl
from jax.experimental.pallas import tpu as pltpu
from jax.experimental.pallas import tpu_sc as plsc

_N = 10000
_NP = 10240
_E = 320000
_NT = 16
_RPT = _NP // _NT
_K = 128
_CPT = 160
_CPG = _NT * _CPT
_EP = _CPG * _K
_NPAIR = _CPT // 2

_mesh = plsc.VectorSubcoreMesh(core_axis_name="c", subcore_axis_name="s")


@functools.partial(
    pl.kernel,
    out_type=jax.ShapeDtypeStruct((2 * _NP,), jnp.float32),
    mesh=_mesh,
    scratch_types=[
        pltpu.VMEM((_CPT, _K), jnp.int32),
        pltpu.VMEM((_RPT,), jnp.float32),
        pltpu.VMEM_SHARED((_NP,), jnp.float32),
    ],
)
def _deg(dst_hbm, deg_hbm, dst_blk, ones_v, acc):
    c = lax.axis_index("c")
    s = lax.axis_index("s")

    def fill(i, carry):
        ones_v[pl.ds(i * 16, 16)] = jnp.full((16,), 1.0, jnp.float32)
        return carry

    lax.fori_loop(0, _RPT // 16, fill, 0)
    pltpu.sync_copy(dst_hbm.at[pl.ds(c * _CPG + s * _CPT, _CPT)], dst_blk)
    pltpu.sync_copy(ones_v, acc.at[pl.ds(s * _RPT, _RPT)])
    plsc.subcore_barrier()

    def body(i, carry):
        pltpu.sync_copy(ones_v.at[pl.ds(0, _K)], acc.at[dst_blk.at[i]],
                        add=True)
        return carry

    lax.fori_loop(0, _CPT, body, 0)
    plsc.subcore_barrier()
    pltpu.sync_copy(acc.at[pl.ds(s * _RPT, _RPT)], ones_v)
    pltpu.sync_copy(ones_v, deg_hbm.at[pl.ds(c * _NP + s * _RPT, _RPT)])


_BPH = 40
_NPH = _CPT // _BPH


@functools.partial(
    pl.kernel,
    out_type=jax.ShapeDtypeStruct((2 * _NP, 128), jnp.float32),
    mesh=_mesh,
    scratch_types=[
        pltpu.VMEM((_BPH, _K), jnp.int32),
        pltpu.VMEM((_BPH, _K), jnp.int32),
        pltpu.VMEM((_K, 128), jnp.float32),
        pltpu.VMEM((_K, 128), jnp.float32),
        pltpu.VMEM_SHARED((_NP, 128), jnp.float32),
        pltpu.SemaphoreType.DMA,
        pltpu.SemaphoreType.DMA,
    ],
)
def _agg(hp_hbm, src_hbm, dst_hbm, out_hbm, src_blk, dst_blk,
         rows0, rows1, acc, gsem0, gsem1):
    c = lax.axis_index("c")
    s = lax.axis_index("s")
    row0 = s * _RPT
    erow0 = c * _CPG + s * _CPT
    for k in range(_RPT // _K):
        pltpu.sync_copy(hp_hbm.at[pl.ds(c * _NP + row0 + k * _K, _K)], rows0)
        pltpu.sync_copy(rows0, acc.at[pl.ds(row0 + k * _K, _K)])
    plsc.subcore_barrier()

    rows = (rows0, rows1)
    gsems = (gsem0, gsem1)

    def _gather(ii, p):
        pltpu.async_copy(hp_hbm.at[src_blk.at[ii]], rows[p], gsems[p])

    def _chunk(ii, p, last):
        if not last:
            _gather(ii + 1, 1 - p)
        pltpu.make_async_copy(hp_hbm.at[src_blk.at[ii]], rows[p],
                              gsems[p]).wait()
        pltpu.sync_copy(rows[p], acc.at[dst_blk.at[ii]], add=True)

    for phase in range(_NPH):
        pltpu.sync_copy(src_hbm.at[pl.ds(erow0 + phase * _BPH, _BPH)],
                        src_blk)
        pltpu.sync_copy(dst_hbm.at[pl.ds(erow0 + phase * _BPH, _BPH)],
                        dst_blk)
        _gather(0, 0)

        def pair(j, carry):
            _chunk(2 * j, 0, False)
            _chunk(2 * j + 1, 1, False)
            return carry

        lax.fori_loop(0, _BPH // 2 - 1, pair, 0)
        _chunk(_BPH - 2, 0, False)
        _chunk(_BPH - 1, 1, True)

    plsc.subcore_barrier()
    for k in range(_RPT // _K):
        pltpu.sync_copy(acc.at[pl.ds(row0 + k * _K, _K)], rows0)
        pltpu.sync_copy(rows0, out_hbm.at[pl.ds(c * _NP + row0 + k * _K,
                                                _K)])


_BM = 2048


def _tc1(x, W, deg):
    def body(x_ref, w_ref, deg_ref, hp_ref, dinv_ref):
        d = lax.rsqrt(deg_ref[...])
        h = jnp.dot(x_ref[...], w_ref[...], preferred_element_type=jnp.float32)
        hp_ref[...] = d * h
        dinv_ref[...] = d

    return pl.pallas_call(
        body,
        grid=(2 * _NP // _BM,),
        in_specs=[
            pl.BlockSpec((_BM, 128), lambda i: (i, 0)),
            pl.BlockSpec((128, 128), lambda i: (0, 0)),
            pl.BlockSpec((_BM, 1), lambda i: (i, 0)),
        ],
        out_specs=[
            pl.BlockSpec((_BM, 128), lambda i: (i, 0)),
            pl.BlockSpec((_BM, 1), lambda i: (i, 0)),
        ],
        out_shape=[
            jax.ShapeDtypeStruct((2 * _NP, 128), jnp.float32),
            jax.ShapeDtypeStruct((2 * _NP, 1), jnp.float32),
        ],
    )(x, W, deg)


def _tc2(agg, dinv, b, W):
    F = b.shape[1]

    def body(a_ref, d_ref, b_ref, w_ref, o_ref):
        d = d_ref[...]
        y = jnp.maximum(d * a_ref[:, :F] + b_ref[...], 0.0)
        o_ref[...] = d * jnp.dot(y, w_ref[...],
                                 preferred_element_type=jnp.float32)

    return pl.pallas_call(
        body,
        grid=(2 * _NP // _BM,),
        in_specs=[
            pl.BlockSpec((_BM, 128), lambda i: (i, 0)),
            pl.BlockSpec((_BM, 1), lambda i: (i, 0)),
            pl.BlockSpec((1, F), lambda i: (0, 0)),
            pl.BlockSpec((F, 128), lambda i: (0, 0)),
        ],
        out_specs=pl.BlockSpec((_BM, 128), lambda i: (i, 0)),
        out_shape=jax.ShapeDtypeStruct((2 * _NP, 128), jnp.float32),
    )(agg, dinv, b, W)


def _head(agg3, dinv, b3, att_W, ntn_Wt, ntn_Vt, ntn_bR, f1w, f1b, f2w, f2b,
          f3w, f3b, scw, scb, av):
    def body(a_ref, d_ref, b3_ref, aw_ref, nw_ref, nv_ref, nb_ref, f1w_ref,
             f1b_ref, f2w_ref, f2b_ref, f3w_ref, f3b_ref, scw_ref, scb_ref,
             av_ref, score_ref, ged_ref):
        ps = []
        for g in range(2):
            y = (d_ref[pl.ds(g * _NP, _N), :]
                 * a_ref[pl.ds(g * _NP, _N), :32] + b3_ref[...])
            t1 = jnp.dot(y, aw_ref[...], preferred_element_type=jnp.float32)
            gc = jnp.sum(t1, axis=0, keepdims=True) * (1.0 / _N)
            tg = jnp.tanh(gc)
            sall = jax.nn.sigmoid(jnp.sum(y * tg, axis=1, keepdims=True))
            ps.append(jnp.sum(y * sall, axis=0, keepdims=True))
        p1, p2 = ps
        sc_list = []
        for t in range(16):
            m = jnp.sum(nw_ref[t] * p2, axis=1, keepdims=True)
            sc_list.append(jnp.dot(p1, m, preferred_element_type=jnp.float32))
        scoring = jnp.concatenate(sc_list, axis=1)
        combined = jnp.concatenate([p1, p2], axis=1)
        block = jnp.dot(combined, nv_ref[...],
                        preferred_element_type=jnp.float32)
        scores = jnp.maximum(scoring + block + nb_ref[...], 0.0)
        h = jnp.maximum(jnp.dot(scores, f1w_ref[...],
                                preferred_element_type=jnp.float32)
                        + f1b_ref[...], 0.0)
        h = jnp.maximum(jnp.dot(h, f2w_ref[...],
                                preferred_element_type=jnp.float32)
                        + f2b_ref[...], 0.0)
        h = jnp.maximum(jnp.dot(h, f3w_ref[...],
                                preferred_element_type=jnp.float32)
                        + f3b_ref[...], 0.0)
        sc = jax.nn.sigmoid(jnp.dot(h, scw_ref[...],
                                    preferred_element_type=jnp.float32)
                            + scb_ref[...])
        score_ref[...] = sc
        ged_ref[...] = -jnp.log(sc) * av_ref[...]

    return pl.pallas_call(
        body,
        out_shape=[
            jax.ShapeDtypeStruct((1, 1), jnp.float32),
            jax.ShapeDtypeStruct((1, 1), jnp.float32),
        ],
    )(agg3, dinv, b3, att_W, ntn_Wt, ntn_Vt, ntn_bR, f1w, f1b, f2w, f2b,
      f3w, f3b, scw, scb, av)


def kernel(features_1, features_2, edge_index_1, edge_index_2, avg_v,
           W1, b1, W2, b2, W3, b3, att_W, ntn_W, ntn_V, ntn_b,
           fc1_W, fc1_b, fc2_W, fc2_b, fc3_W, fc3_b, sc_W, sc_b):
    pad = jnp.zeros((_NP - _N, 128), jnp.float32)
    x = jnp.concatenate([features_1, pad, features_2, pad], axis=0)
    epad = jnp.full((_EP - _E,), _N, jnp.int32) + (
        jnp.arange(_EP - _E, dtype=jnp.int32) % (_NP - _N))
    src = jnp.concatenate([edge_index_1[0], epad,
                           edge_index_2[0] + _NP, epad + _NP])
    dst = jnp.concatenate([edge_index_1[1], epad, edge_index_2[1], epad])

    src = src.reshape(2 * _CPG, _K)
    dst = dst.reshape(2 * _CPG, _K)

    deg = _deg(dst).reshape(2 * _NP, 1)

    W2p = jnp.pad(W2, ((0, 0), (0, 128 - W2.shape[1])))
    W3p = jnp.pad(W3, ((0, 0), (0, 128 - W3.shape[1])))

    hp1, dinv = _tc1(x, W1, deg)
    agg1 = _agg(hp1, src, dst)
    hp2 = _tc2(agg1, dinv, b1.reshape(1, -1), W2p)
    agg2 = _agg(hp2, src, dst)
    hp3 = _tc2(agg2, dinv, b2.reshape(1, -1), W3p)
    agg3 = _agg(hp3, src, dst)

    score, ged = _head(
        agg3, dinv, b3.reshape(1, -1), att_W,
        jnp.transpose(ntn_W, (2, 0, 1)), ntn_V.T, ntn_b.reshape(1, -1),
        fc1_W, fc1_b.reshape(1, -1), fc2_W, fc2_b.reshape(1, -1),
        fc3_W, fc3_b.reshape(1, -1), sc_W, sc_b.reshape(1, -1),
        avg_v.reshape(1, 1))
    return score.reshape(-1), ged.reshape(-1)


# speedup vs baseline: 1.2715x; 1.0138x over previous
import functools
import jax
import jax.numpy as jnp
from jax import lax
from jax.experimental import pallas as pl
from jax.experimental.pallas import tpu as pltpu
from jax.experimental.pallas import tpu_sc as plsc

_N = 10000
_NP = 10240
_E = 320000
_NT = 16
_RPT = _NP // _NT
_K = 128
_CPT = 160
_CPG = _NT * _CPT
_EP = _CPG * _K
_NPAIR = _CPT // 2

_mesh = plsc.VectorSubcoreMesh(core_axis_name="c", subcore_axis_name="s")


@functools.partial(
    pl.kernel,
    out_type=jax.ShapeDtypeStruct((2 * _NP,), jnp.float32),
    mesh=_mesh,
    scratch_types=[
        pltpu.VMEM((_CPT, _K), jnp.int32),
        pltpu.VMEM((_RPT,), jnp.float32),
        pltpu.VMEM_SHARED((_NP,), jnp.float32),
    ],
)
def _deg(dst_hbm, deg_hbm, dst_blk, ones_v, acc):
    c = lax.axis_index("c")
    s = lax.axis_index("s")

    def fill(i, carry):
        ones_v[pl.ds(i * 16, 16)] = jnp.full((16,), 1.0, jnp.float32)
        return carry

    lax.fori_loop(0, _RPT // 16, fill, 0)
    pltpu.sync_copy(dst_hbm.at[pl.ds(c * _CPG + s * _CPT, _CPT)], dst_blk)
    pltpu.sync_copy(ones_v, acc.at[pl.ds(s * _RPT, _RPT)])
    plsc.subcore_barrier()

    def body(i, carry):
        pltpu.sync_copy(ones_v.at[pl.ds(0, _K)], acc.at[dst_blk.at[i]],
                        add=True)
        return carry

    lax.fori_loop(0, _CPT, body, 0)
    plsc.subcore_barrier()
    pltpu.sync_copy(acc.at[pl.ds(s * _RPT, _RPT)], ones_v)
    pltpu.sync_copy(ones_v, deg_hbm.at[pl.ds(c * _NP + s * _RPT, _RPT)])


_BPH = 16
_NPH = _CPT // _BPH


@functools.partial(
    pl.kernel,
    out_type=jax.ShapeDtypeStruct((2 * _NP, 128), jnp.float32),
    mesh=_mesh,
    scratch_types=[
        pltpu.VMEM((_BPH, _K), jnp.int32),
        pltpu.VMEM((_BPH, _K), jnp.int32),
        pltpu.VMEM((_BPH, _K), jnp.int32),
        pltpu.VMEM((_BPH, _K), jnp.int32),
        pltpu.VMEM((_K, 128), jnp.float32),
        pltpu.VMEM((_K, 128), jnp.float32),
        pltpu.VMEM_SHARED((_NP, 128), jnp.float32),
        pltpu.SemaphoreType.DMA,
        pltpu.SemaphoreType.DMA,
        pltpu.SemaphoreType.DMA,
        pltpu.SemaphoreType.DMA,
    ],
)
def _agg(hp_hbm, src_hbm, dst_hbm, out_hbm, src_a, dst_a, src_b, dst_b,
         rows0, rows1, acc, isem_a, isem_b, gsem0, gsem1):
    c = lax.axis_index("c")
    s = lax.axis_index("s")
    row0 = s * _RPT
    erow0 = c * _CPG + s * _CPT
    for k in range(_RPT // _K):
        pltpu.sync_copy(hp_hbm.at[pl.ds(c * _NP + row0 + k * _K, _K)], rows0)
        pltpu.sync_copy(rows0, acc.at[pl.ds(row0 + k * _K, _K)])
    plsc.subcore_barrier()

    rows = (rows0, rows1)
    gsems = (gsem0, gsem1)
    blks = ((src_a, dst_a), (src_b, dst_b))
    isems = (isem_a, isem_b)

    def _blk_load(q, p, sync=False):
        sl = pl.ds(erow0 + q * _BPH, _BPH)
        if sync:
            pltpu.sync_copy(src_hbm.at[sl], blks[p][0])
            pltpu.sync_copy(dst_hbm.at[sl], blks[p][1])
        else:
            pltpu.async_copy(src_hbm.at[sl], blks[p][0], isems[p])
            pltpu.async_copy(dst_hbm.at[sl], blks[p][1], isems[p])

    def _blk_wait(q, p):
        sl = pl.ds(erow0 + q * _BPH, _BPH)
        pltpu.make_async_copy(src_hbm.at[sl], blks[p][0], isems[p]).wait()
        pltpu.make_async_copy(dst_hbm.at[sl], blks[p][1], isems[p]).wait()

    def _gather(srcb, ii, r):
        pltpu.async_copy(hp_hbm.at[srcb.at[ii]], rows[r], gsems[r])

    def _retire(srcb, dstb, ii, r):
        pltpu.make_async_copy(hp_hbm.at[srcb.at[ii]], rows[r],
                              gsems[r]).wait()
        pltpu.sync_copy(rows[r], acc.at[dstb.at[ii]], add=True)

    _blk_load(0, 0, sync=True)
    _blk_load(1, 1)
    _gather(src_a, 0, 0)

    for q in range(_NPH):
        sb, db = blks[q % 2]
        nsb = blks[1 - q % 2][0]

        def pair(j, carry, sb=sb, db=db):
            _gather(sb, 2 * j + 1, 1)
            _retire(sb, db, 2 * j, 0)
            _gather(sb, 2 * j + 2, 0)
            _retire(sb, db, 2 * j + 1, 1)
            return carry

        lax.fori_loop(0, _BPH // 2 - 1, pair, 0)
        _gather(sb, _BPH - 1, 1)
        _retire(sb, db, _BPH - 2, 0)
        if q < _NPH - 1:
            _blk_wait(q + 1, 1 - q % 2)
            _gather(nsb, 0, 0)
        _retire(sb, db, _BPH - 1, 1)
        if q + 2 < _NPH:
            _blk_load(q + 2, q % 2)

    plsc.subcore_barrier()
    for k in range(_RPT // _K):
        pltpu.sync_copy(acc.at[pl.ds(row0 + k * _K, _K)], rows0)
        pltpu.sync_copy(rows0, out_hbm.at[pl.ds(c * _NP + row0 + k * _K,
                                                _K)])


_BM = 2048


def _tc1(x, W, deg):
    def body(x_ref, w_ref, deg_ref, hp_ref, dinv_ref):
        d = lax.rsqrt(deg_ref[...])
        h = jnp.dot(x_ref[...], w_ref[...], preferred_element_type=jnp.float32)
        hp_ref[...] = d * h
        dinv_ref[...] = d

    return pl.pallas_call(
        body,
        grid=(2 * _NP // _BM,),
        in_specs=[
            pl.BlockSpec((_BM, 128), lambda i: (i, 0)),
            pl.BlockSpec((128, 128), lambda i: (0, 0)),
            pl.BlockSpec((_BM, 1), lambda i: (i, 0)),
        ],
        out_specs=[
            pl.BlockSpec((_BM, 128), lambda i: (i, 0)),
            pl.BlockSpec((_BM, 1), lambda i: (i, 0)),
        ],
        out_shape=[
            jax.ShapeDtypeStruct((2 * _NP, 128), jnp.float32),
            jax.ShapeDtypeStruct((2 * _NP, 1), jnp.float32),
        ],
    )(x, W, deg)


def _tc2(agg, dinv, b, W):
    F = b.shape[1]

    def body(a_ref, d_ref, b_ref, w_ref, o_ref):
        d = d_ref[...]
        y = jnp.maximum(d * a_ref[:, :F] + b_ref[...], 0.0)
        o_ref[...] = d * jnp.dot(y, w_ref[...],
                                 preferred_element_type=jnp.float32)

    return pl.pallas_call(
        body,
        grid=(2 * _NP // _BM,),
        in_specs=[
            pl.BlockSpec((_BM, 128), lambda i: (i, 0)),
            pl.BlockSpec((_BM, 1), lambda i: (i, 0)),
            pl.BlockSpec((1, F), lambda i: (0, 0)),
            pl.BlockSpec((F, 128), lambda i: (0, 0)),
        ],
        out_specs=pl.BlockSpec((_BM, 128), lambda i: (i, 0)),
        out_shape=jax.ShapeDtypeStruct((2 * _NP, 128), jnp.float32),
    )(agg, dinv, b, W)


def _head(agg3, dinv, b3, att_W, ntn_Wt, ntn_Vt, ntn_bR, f1w, f1b, f2w, f2b,
          f3w, f3b, scw, scb, av):
    def body(a_ref, d_ref, b3_ref, aw_ref, nw_ref, nv_ref, nb_ref, f1w_ref,
             f1b_ref, f2w_ref, f2b_ref, f3w_ref, f3b_ref, scw_ref, scb_ref,
             av_ref, score_ref, ged_ref):
        ps = []
        for g in range(2):
            y = (d_ref[pl.ds(g * _NP, _N), :]
                 * a_ref[pl.ds(g * _NP, _N), :32] + b3_ref[...])
            t1 = jnp.dot(y, aw_ref[...], preferred_element_type=jnp.float32)
            gc = jnp.sum(t1, axis=0, keepdims=True) * (1.0 / _N)
            tg = jnp.tanh(gc)
            sall = jax.nn.sigmoid(jnp.sum(y * tg, axis=1, keepdims=True))
            ps.append(jnp.sum(y * sall, axis=0, keepdims=True))
        p1, p2 = ps
        sc_list = []
        for t in range(16):
            m = jnp.sum(nw_ref[t] * p2, axis=1, keepdims=True)
            sc_list.append(jnp.dot(p1, m, preferred_element_type=jnp.float32))
        scoring = jnp.concatenate(sc_list, axis=1)
        combined = jnp.concatenate([p1, p2], axis=1)
        block = jnp.dot(combined, nv_ref[...],
                        preferred_element_type=jnp.float32)
        scores = jnp.maximum(scoring + block + nb_ref[...], 0.0)
        h = jnp.maximum(jnp.dot(scores, f1w_ref[...],
                                preferred_element_type=jnp.float32)
                        + f1b_ref[...], 0.0)
        h = jnp.maximum(jnp.dot(h, f2w_ref[...],
                                preferred_element_type=jnp.float32)
                        + f2b_ref[...], 0.0)
        h = jnp.maximum(jnp.dot(h, f3w_ref[...],
                                preferred_element_type=jnp.float32)
                        + f3b_ref[...], 0.0)
        sc = jax.nn.sigmoid(jnp.dot(h, scw_ref[...],
                                    preferred_element_type=jnp.float32)
                            + scb_ref[...])
        score_ref[...] = sc
        ged_ref[...] = -jnp.log(sc) * av_ref[...]

    return pl.pallas_call(
        body,
        out_shape=[
            jax.ShapeDtypeStruct((1, 1), jnp.float32),
            jax.ShapeDtypeStruct((1, 1), jnp.float32),
        ],
    )(agg3, dinv, b3, att_W, ntn_Wt, ntn_Vt, ntn_bR, f1w, f1b, f2w, f2b,
      f3w, f3b, scw, scb, av)


def kernel(features_1, features_2, edge_index_1, edge_index_2, avg_v,
           W1, b1, W2, b2, W3, b3, att_W, ntn_W, ntn_V, ntn_b,
           fc1_W, fc1_b, fc2_W, fc2_b, fc3_W, fc3_b, sc_W, sc_b):
    pad = jnp.zeros((_NP - _N, 128), jnp.float32)
    x = jnp.concatenate([features_1, pad, features_2, pad], axis=0)
    epad = jnp.full((_EP - _E,), _N, jnp.int32) + (
        jnp.arange(_EP - _E, dtype=jnp.int32) % (_NP - _N))
    src = jnp.concatenate([edge_index_1[0], epad,
                           edge_index_2[0] + _NP, epad + _NP])
    dst = jnp.concatenate([edge_index_1[1], epad, edge_index_2[1], epad])

    src = src.reshape(2 * _CPG, _K)
    dst = dst.reshape(2 * _CPG, _K)

    deg = _deg(dst).reshape(2 * _NP, 1)

    W2p = jnp.pad(W2, ((0, 0), (0, 128 - W2.shape[1])))
    W3p = jnp.pad(W3, ((0, 0), (0, 128 - W3.shape[1])))

    hp1, dinv = _tc1(x, W1, deg)
    agg1 = _agg(hp1, src, dst)
    hp2 = _tc2(agg1, dinv, b1.reshape(1, -1), W2p)
    agg2 = _agg(hp2, src, dst)
    hp3 = _tc2(agg2, dinv, b2.reshape(1, -1), W3p)
    agg3 = _agg(hp3, src, dst)

    score, ged = _head(
        agg3, dinv, b3.reshape(1, -1), att_W,
        jnp.transpose(ntn_W, (2, 0, 1)), ntn_V.T, ntn_b.reshape(1, -1),
        fc1_W, fc1_b.reshape(1, -1), fc2_W, fc2_b.reshape(1, -1),
        fc3_W, fc3_b.reshape(1, -1), sc_W, sc_b.reshape(1, -1),
        avg_v.reshape(1, 1))
    return score.reshape(-1), ged.reshape(-1)


# speedup vs baseline: 1.2774x; 1.0047x over previous
import functools
import jax
import jax.numpy as jnp
from jax import lax
from jax.experimental import pallas as pl
from jax.experimental.pallas import tpu as pltpu
from jax.experimental.pallas import tpu_sc as plsc

_N = 10000
_NP = 10240
_E = 320000
_NT = 16
_RPT = _NP // _NT
_K = 128
_CPT = 160
_CPG = _NT * _CPT
_EP = _CPG * _K

_mesh = plsc.VectorSubcoreMesh(core_axis_name="c", subcore_axis_name="s")


@functools.partial(
    pl.kernel,
    out_type=jax.ShapeDtypeStruct((2 * _NP,), jnp.float32),
    mesh=_mesh,
    scratch_types=[
        pltpu.VMEM((_CPT, _K), jnp.int32),
        pltpu.VMEM((_RPT,), jnp.float32),
        pltpu.VMEM_SHARED((_NP,), jnp.float32),
    ],
)
def _deg(dst_hbm, deg_hbm, dst_blk, ones_v, acc):
    c = lax.axis_index("c")
    s = lax.axis_index("s")

    def fill(i, carry):
        ones_v[pl.ds(i * 16, 16)] = jnp.full((16,), 1.0, jnp.float32)
        return carry

    lax.fori_loop(0, _RPT // 16, fill, 0)
    pltpu.sync_copy(dst_hbm.at[pl.ds(c * _CPG + s * _CPT, _CPT)], dst_blk)
    pltpu.sync_copy(ones_v, acc.at[pl.ds(s * _RPT, _RPT)])
    plsc.subcore_barrier()

    def body(i, carry):
        pltpu.sync_copy(ones_v.at[pl.ds(0, _K)], acc.at[dst_blk.at[i]],
                        add=True)
        return carry

    lax.fori_loop(0, _CPT, body, 0)
    plsc.subcore_barrier()
    pltpu.sync_copy(acc.at[pl.ds(s * _RPT, _RPT)], ones_v)
    pltpu.sync_copy(ones_v, deg_hbm.at[pl.ds(c * _NP + s * _RPT, _RPT)])


_BPH = 16
_NPH = _CPT // _BPH


@functools.partial(
    pl.kernel,
    out_type=jax.ShapeDtypeStruct((2 * _NP, 128), jnp.float32),
    mesh=_mesh,
    scratch_types=[
        pltpu.VMEM((_BPH, _K), jnp.int32),
        pltpu.VMEM((_BPH, _K), jnp.int32),
        pltpu.VMEM((_BPH, _K), jnp.int32),
        pltpu.VMEM((_BPH, _K), jnp.int32),
        pltpu.VMEM((_K, 128), jnp.float32),
        pltpu.VMEM((_K, 128), jnp.float32),
        pltpu.VMEM_SHARED((_NP, 128), jnp.float32),
        pltpu.SemaphoreType.DMA,
        pltpu.SemaphoreType.DMA,
        pltpu.SemaphoreType.DMA,
        pltpu.SemaphoreType.DMA,
    ],
)
def _agg(hp_hbm, src_hbm, dst_hbm, out_hbm, src_a, dst_a, src_b, dst_b,
         rows0, rows1, acc, isem_a, isem_b, gsem0, gsem1):
    c = lax.axis_index("c")
    s = lax.axis_index("s")
    row0 = s * _RPT
    erow0 = c * _CPG + s * _CPT
    for k in range(_RPT // _K):
        pltpu.sync_copy(hp_hbm.at[pl.ds(c * _NP + row0 + k * _K, _K)], rows0)
        pltpu.sync_copy(rows0, acc.at[pl.ds(row0 + k * _K, _K)])
    plsc.subcore_barrier()

    rows = (rows0, rows1)
    gsems = (gsem0, gsem1)
    blks = ((src_a, dst_a), (src_b, dst_b))
    isems = (isem_a, isem_b)

    def _blk_load(q, p, sync=False):
        sl = pl.ds(erow0 + q * _BPH, _BPH)
        if sync:
            pltpu.sync_copy(src_hbm.at[sl], blks[p][0])
            pltpu.sync_copy(dst_hbm.at[sl], blks[p][1])
        else:
            pltpu.async_copy(src_hbm.at[sl], blks[p][0], isems[p])
            pltpu.async_copy(dst_hbm.at[sl], blks[p][1], isems[p])

    def _blk_wait(q, p):
        sl = pl.ds(erow0 + q * _BPH, _BPH)
        pltpu.make_async_copy(src_hbm.at[sl], blks[p][0], isems[p]).wait()
        pltpu.make_async_copy(dst_hbm.at[sl], blks[p][1], isems[p]).wait()

    def _gather(srcb, ii, r):
        pltpu.async_copy(hp_hbm.at[srcb.at[ii]], rows[r], gsems[r])

    def _retire(srcb, dstb, ii, r):
        pltpu.make_async_copy(hp_hbm.at[srcb.at[ii]], rows[r],
                              gsems[r]).wait()
        pltpu.sync_copy(rows[r], acc.at[dstb.at[ii]], add=True)

    _blk_load(0, 0, sync=True)
    _blk_load(1, 1)
    _gather(src_a, 0, 0)

    for q in range(_NPH):
        sb, db = blks[q % 2]
        nsb = blks[1 - q % 2][0]

        def pair(j, carry, sb=sb, db=db):
            _gather(sb, 2 * j + 1, 1)
            _retire(sb, db, 2 * j, 0)
            _gather(sb, 2 * j + 2, 0)
            _retire(sb, db, 2 * j + 1, 1)
            return carry

        lax.fori_loop(0, _BPH // 2 - 1, pair, 0)
        _gather(sb, _BPH - 1, 1)
        _retire(sb, db, _BPH - 2, 0)
        if q < _NPH - 1:
            _blk_wait(q + 1, 1 - q % 2)
            _gather(nsb, 0, 0)
        _retire(sb, db, _BPH - 1, 1)
        if q + 2 < _NPH:
            _blk_load(q + 2, q % 2)

    plsc.subcore_barrier()
    for k in range(_RPT // _K):
        pltpu.sync_copy(acc.at[pl.ds(row0 + k * _K, _K)], rows0)
        pltpu.sync_copy(rows0, out_hbm.at[pl.ds(c * _NP + row0 + k * _K,
                                                _K)])


_BM = 2048


def _tc1(x, W, deg):
    def body(x_ref, w_ref, deg_ref, hp_ref, dinv_ref):
        d = lax.rsqrt(deg_ref[...])
        h = jnp.dot(x_ref[...], w_ref[...], preferred_element_type=jnp.float32)
        hp_ref[...] = d * h
        dinv_ref[...] = d

    return pl.pallas_call(
        body,
        grid=(2 * _NP // _BM,),
        in_specs=[
            pl.BlockSpec((_BM, 128), lambda i: (i, 0)),
            pl.BlockSpec((128, 128), lambda i: (0, 0)),
            pl.BlockSpec((_BM, 1), lambda i: (i, 0)),
        ],
        out_specs=[
            pl.BlockSpec((_BM, 128), lambda i: (i, 0)),
            pl.BlockSpec((_BM, 1), lambda i: (i, 0)),
        ],
        out_shape=[
            jax.ShapeDtypeStruct((2 * _NP, 128), jnp.float32),
            jax.ShapeDtypeStruct((2 * _NP, 1), jnp.float32),
        ],
    )(x, W, deg)


def _tc2(agg, dinv, b, W):
    F = b.shape[1]

    def body(a_ref, d_ref, b_ref, w_ref, o_ref):
        d = d_ref[...]
        y = jnp.maximum(d * a_ref[:, :F] + b_ref[...], 0.0)
        o_ref[...] = d * jnp.dot(y, w_ref[...],
                                 preferred_element_type=jnp.float32)

    return pl.pallas_call(
        body,
        grid=(2 * _NP // _BM,),
        in_specs=[
            pl.BlockSpec((_BM, 128), lambda i: (i, 0)),
            pl.BlockSpec((_BM, 1), lambda i: (i, 0)),
            pl.BlockSpec((1, F), lambda i: (0, 0)),
            pl.BlockSpec((F, 128), lambda i: (0, 0)),
        ],
        out_specs=pl.BlockSpec((_BM, 128), lambda i: (i, 0)),
        out_shape=jax.ShapeDtypeStruct((2 * _NP, 128), jnp.float32),
    )(agg, dinv, b, W)


def _head(agg3, dinv, b3, att_W, ntn_Wt, ntn_Vt, ntn_bR, f1w, f1b, f2w, f2b,
          f3w, f3b, scw, scb, av):
    def body(a_ref, d_ref, b3_ref, aw_ref, nw_ref, nv_ref, nb_ref, f1w_ref,
             f1b_ref, f2w_ref, f2b_ref, f3w_ref, f3b_ref, scw_ref, scb_ref,
             av_ref, score_ref, ged_ref):
        ps = []
        for g in range(2):
            y = (d_ref[pl.ds(g * _NP, _N), :]
                 * a_ref[pl.ds(g * _NP, _N), :32] + b3_ref[...])
            t1 = jnp.dot(y, aw_ref[...], preferred_element_type=jnp.float32)
            gc = jnp.sum(t1, axis=0, keepdims=True) * (1.0 / _N)
            tg = jnp.tanh(gc)
            sall = jax.nn.sigmoid(jnp.sum(y * tg, axis=1, keepdims=True))
            ps.append(jnp.sum(y * sall, axis=0, keepdims=True))
        p1, p2 = ps
        sc_list = []
        for t in range(16):
            m = jnp.sum(nw_ref[t] * p2, axis=1, keepdims=True)
            sc_list.append(jnp.dot(p1, m, preferred_element_type=jnp.float32))
        scoring = jnp.concatenate(sc_list, axis=1)
        combined = jnp.concatenate([p1, p2], axis=1)
        block = jnp.dot(combined, nv_ref[...],
                        preferred_element_type=jnp.float32)
        scores = jnp.maximum(scoring + block + nb_ref[...], 0.0)
        h = jnp.maximum(jnp.dot(scores, f1w_ref[...],
                                preferred_element_type=jnp.float32)
                        + f1b_ref[...], 0.0)
        h = jnp.maximum(jnp.dot(h, f2w_ref[...],
                                preferred_element_type=jnp.float32)
                        + f2b_ref[...], 0.0)
        h = jnp.maximum(jnp.dot(h, f3w_ref[...],
                                preferred_element_type=jnp.float32)
                        + f3b_ref[...], 0.0)
        sc = jax.nn.sigmoid(jnp.dot(h, scw_ref[...],
                                    preferred_element_type=jnp.float32)
                            + scb_ref[...])
        score_ref[...] = sc
        ged_ref[...] = -jnp.log(sc) * av_ref[...]

    return pl.pallas_call(
        body,
        out_shape=[
            jax.ShapeDtypeStruct((1, 1), jnp.float32),
            jax.ShapeDtypeStruct((1, 1), jnp.float32),
        ],
    )(agg3, dinv, b3, att_W, ntn_Wt, ntn_Vt, ntn_bR, f1w, f1b, f2w, f2b,
      f3w, f3b, scw, scb, av)


def kernel(features_1, features_2, edge_index_1, edge_index_2, avg_v,
           W1, b1, W2, b2, W3, b3, att_W, ntn_W, ntn_V, ntn_b,
           fc1_W, fc1_b, fc2_W, fc2_b, fc3_W, fc3_b, sc_W, sc_b):
    pad = jnp.zeros((_NP - _N, 128), jnp.float32)
    x = jnp.concatenate([features_1, pad, features_2, pad], axis=0)
    epad = jnp.full((_EP - _E,), _N, jnp.int32) + (
        jnp.arange(_EP - _E, dtype=jnp.int32) % (_NP - _N))
    src = jnp.concatenate([edge_index_1[0], epad,
                           edge_index_2[0] + _NP, epad + _NP])
    dst = jnp.concatenate([edge_index_1[1], epad, edge_index_2[1], epad])

    src = src.reshape(2 * _CPG, _K)
    dst = dst.reshape(2 * _CPG, _K)

    deg = _deg(dst).reshape(2 * _NP, 1)

    W2p = jnp.pad(W2, ((0, 0), (0, 128 - W2.shape[1])))
    W3p = jnp.pad(W3, ((0, 0), (0, 128 - W3.shape[1])))

    hp1, dinv = _tc1(x, W1, deg)
    agg1 = _agg(hp1, src, dst)
    hp2 = _tc2(agg1, dinv, b1.reshape(1, -1), W2p)
    agg2 = _agg(hp2, src, dst)
    hp3 = _tc2(agg2, dinv, b2.reshape(1, -1), W3p)
    agg3 = _agg(hp3, src, dst)

    score, ged = _head(
        agg3, dinv, b3.reshape(1, -1), att_W,
        jnp.transpose(ntn_W, (2, 0, 1)), ntn_V.T, ntn_b.reshape(1, -1),
        fc1_W, fc1_b.reshape(1, -1), fc2_W, fc2_b.reshape(1, -1),
        fc3_W, fc3_b.reshape(1, -1), sc_W, sc_b.reshape(1, -1),
        avg_v.reshape(1, 1))
    return score.reshape(-1), ged.reshape(-1)
